# baseline (device time: 106342 ns/iter reference)
import jax
import jax.numpy as jnp
from jax import lax
from jax.experimental import pallas as pl
from jax.experimental.pallas import tpu as pltpu

N_DEV = 4
KC = 256


def kernel(x, W1, W2):
    m, k_in = x.shape
    _, h_per = W1.shape
    _, n = W2.shape
    nk = h_per // KC
    m2 = m // 2
    cm = m2 // N_DEV

    def body(x_ref, w1_hbm, w2_hbm, out_ref,
             x_bf, h_bf, w2_bf, w1_buf, w2_buf, load_sems,
             pacc, comm_ref, send_sems, recv_sems):
        my = lax.axis_index("i")
        left = (my + N_DEV - 1) % N_DEV
        right = (my + 1) % N_DEV

        barrier_sem = pltpu.get_barrier_semaphore()
        for nbr in (left, right):
            pl.semaphore_signal(
                barrier_sem, inc=1,
                device_id=(nbr,), device_id_type=pl.DeviceIdType.MESH,
            )
        pl.semaphore_wait(barrier_sem, 2)

        def load(kc, slot):
            c1 = pltpu.make_async_copy(
                w1_hbm.at[:, pl.ds(kc * KC, KC)], w1_buf.at[slot],
                load_sems.at[slot, 0])
            c2 = pltpu.make_async_copy(
                w2_hbm.at[pl.ds(kc * KC, KC), :], w2_buf.at[slot],
                load_sems.at[slot, 1])
            c1.start()
            c2.start()
            return c1, c2

        pending = load(0, 0)
        x_bf[...] = x_ref[...].astype(jnp.bfloat16)
        for kc in range(nk):
            slot = kc % 2
            if kc + 1 < nk:
                nxt = load(kc + 1, (kc + 1) % 2)
            pending[0].wait()
            pending[1].wait()
            h_bf[:, pl.ds(kc * KC, KC)] = jnp.maximum(
                jnp.dot(x_bf[...], w1_buf[slot].astype(jnp.bfloat16),
                        preferred_element_type=jnp.float32),
                0.0,
            ).astype(jnp.bfloat16)
            w2_bf[pl.ds(kc * KC, KC), :] = w2_buf[slot].astype(jnp.bfloat16)
            if kc + 1 < nk:
                pending = nxt

        dirs = ((1, right, 0), (-1, left, 1))

        def rows(hf, c):
            return pl.ds(hf * m2 + c * cm, cm)

        kb = 512

        def compute_block(hf, c, pacc):
            r = rows(hf, c)
            for j in range(h_per // kb):
                p = jnp.dot(
                    h_bf[r, pl.ds(j * kb, kb)],
                    w2_bf[pl.ds(j * kb, kb), :],
                    preferred_element_type=jnp.float32,
                )
                if j == 0:
                    pacc[...] = p
                else:
                    pacc[...] += p
            out_ref[r, :] = pacc[...]

        for d, tgt, hf in dirs:
            compute_block(hf, my % N_DEV, pacc)
            comm_ref[hf, 0] = out_ref[rows(hf, my % N_DEV), :].astype(
                jnp.bfloat16)
        for s in range(N_DEV - 1):
            ss, rs = s % 2, (s + 1) % 2
            rdmas = []
            for d, tgt, hf in dirs:
                rdma = pltpu.make_async_remote_copy(
                    src_ref=comm_ref.at[hf, ss],
                    dst_ref=comm_ref.at[hf, rs],
                    send_sem=send_sems.at[hf, ss],
                    recv_sem=recv_sems.at[hf, rs],
                    device_id=(tgt,),
                    device_id_type=pl.DeviceIdType.MESH,
                )
                rdma.start()
                rdmas.append(rdma)
            for d, tgt, hf in dirs:
                compute_block(hf, (my - d * (s + 1)) % N_DEV, pacc)
            for rdma in rdmas:
                rdma.wait()
            for d, tgt, hf in dirs:
                recv_c = (my - d * (s + 1)) % N_DEV
                comm_ref[hf, rs] = (
                    comm_ref[hf, rs].astype(jnp.float32)
                    + out_ref[rows(hf, recv_c), :]
                ).astype(jnp.bfloat16)

        own_slot = (N_DEV - 1) % 2
        for d, tgt, hf in dirs:
            out_ref[rows(hf, (my + d) % N_DEV), :] = comm_ref[
                hf, own_slot].astype(jnp.float32)

        for s in range(N_DEV - 1):
            t = (N_DEV - 1) + s
            ss, rs = t % 2, (t + 1) % 2
            rdmas = []
            for d, tgt, hf in dirs:
                rdma = pltpu.make_async_remote_copy(
                    src_ref=comm_ref.at[hf, ss],
                    dst_ref=comm_ref.at[hf, rs],
                    send_sem=send_sems.at[hf, ss],
                    recv_sem=recv_sems.at[hf, rs],
                    device_id=(tgt,),
                    device_id_type=pl.DeviceIdType.MESH,
                )
                rdma.start()
                rdmas.append(rdma)
            for rdma in rdmas:
                rdma.wait()
            for d, tgt, hf in dirs:
                recv_c = (my - d * s) % N_DEV
                out_ref[rows(hf, recv_c), :] = comm_ref[hf, rs].astype(
                    jnp.float32)

    return pl.pallas_call(
        body,
        out_shape=jax.ShapeDtypeStruct((m, n), jnp.float32),
        in_specs=[
            pl.BlockSpec(memory_space=pltpu.VMEM),
            pl.BlockSpec(memory_space=pl.ANY),
            pl.BlockSpec(memory_space=pl.ANY),
        ],
        out_specs=pl.BlockSpec(memory_space=pltpu.VMEM),
        scratch_shapes=[
            pltpu.VMEM((m, k_in), jnp.bfloat16),
            pltpu.VMEM((m, h_per), jnp.bfloat16),
            pltpu.VMEM((h_per, n), jnp.bfloat16),
            pltpu.VMEM((2, k_in, KC), jnp.float32),
            pltpu.VMEM((2, KC, n), jnp.float32),
            pltpu.SemaphoreType.DMA((2, 2)),
            pltpu.VMEM((cm, n), jnp.float32),
            pltpu.VMEM((2, 2, cm, n), jnp.bfloat16),
            pltpu.SemaphoreType.DMA((2, 2)),
            pltpu.SemaphoreType.DMA((2, 2)),
        ],
        compiler_params=pltpu.CompilerParams(
            collective_id=0,
            vmem_limit_bytes=60 * 1024 * 1024,
        ),
    )(x, W1, W2)


# device time: 106157 ns/iter; 1.0017x vs baseline; 1.0017x over previous
import jax
import jax.numpy as jnp
from jax import lax
from jax.experimental import pallas as pl
from jax.experimental.pallas import tpu as pltpu

N_DEV = 4
KC = 256


def kernel(x, W1, W2):
    m, k_in = x.shape
    _, h_per = W1.shape
    _, n = W2.shape
    nk = h_per // KC
    cm = m // N_DEV
    n2 = n // 2

    def body(x_ref, w1_hbm, w2_hbm, out_ref,
             x_bf, h_bf, w2_bf, w1_buf, w2_buf, load_sems,
             comm_ref, send_sems, recv_sems):
        my = lax.axis_index("i")
        left = (my + N_DEV - 1) % N_DEV
        right = (my + 1) % N_DEV

        barrier_sem = pltpu.get_barrier_semaphore()
        for nbr in (left, right):
            pl.semaphore_signal(
                barrier_sem, inc=1,
                device_id=(nbr,), device_id_type=pl.DeviceIdType.MESH,
            )
        pl.semaphore_wait(barrier_sem, 2)

        def load(kc, slot):
            c1 = pltpu.make_async_copy(
                w1_hbm.at[:, pl.ds(kc * KC, KC)], w1_buf.at[slot],
                load_sems.at[slot, 0])
            c2 = pltpu.make_async_copy(
                w2_hbm.at[pl.ds(kc * KC, KC), :], w2_buf.at[slot],
                load_sems.at[slot, 1])
            c1.start()
            c2.start()
            return c1, c2

        pending = load(0, 0)
        x_bf[...] = x_ref[...].astype(jnp.bfloat16)
        for kc in range(nk):
            slot = kc % 2
            if kc + 1 < nk:
                nxt = load(kc + 1, (kc + 1) % 2)
            pending[0].wait()
            pending[1].wait()
            h_bf[:, pl.ds(kc * KC, KC)] = jnp.maximum(
                jnp.dot(x_bf[...], w1_buf[slot].astype(jnp.bfloat16),
                        preferred_element_type=jnp.float32),
                0.0,
            ).astype(jnp.bfloat16)
            w2_bf[pl.ds(kc * KC, KC), :] = w2_buf[slot].astype(jnp.bfloat16)
            if kc + 1 < nk:
                pending = nxt

        dirs = ((1, right, 0), (-1, left, 1))

        def blk(c):
            return pl.ds(c * cm, cm)

        def cols(hf):
            return pl.ds(hf * n2, n2)

        kb = h_per // 2

        def compute_block(c):
            r = blk(c)
            out_ref[r, :] = jnp.dot(
                h_bf[r, pl.ds(0, kb)], w2_bf[pl.ds(0, kb), :],
                preferred_element_type=jnp.float32,
            ) + jnp.dot(
                h_bf[r, pl.ds(kb, kb)], w2_bf[pl.ds(kb, kb), :],
                preferred_element_type=jnp.float32,
            )

        def start_step(t):
            ss, rs = t % 2, (t + 1) % 2
            rdmas = []
            for d, tgt, hf in dirs:
                rdma = pltpu.make_async_remote_copy(
                    src_ref=comm_ref.at[hf, ss],
                    dst_ref=comm_ref.at[hf, rs],
                    send_sem=send_sems.at[hf, ss],
                    recv_sem=recv_sems.at[hf, rs],
                    device_id=(tgt,),
                    device_id_type=pl.DeviceIdType.MESH,
                )
                rdma.start()
                rdmas.append(rdma)
            return rdmas

        compute_block(my % N_DEV)
        for d, tgt, hf in dirs:
            comm_ref[hf, 0] = out_ref[blk(my % N_DEV), cols(hf)].astype(
                jnp.bfloat16)

        for s in range(N_DEV - 1):
            ss, rs = s % 2, (s + 1) % 2
            rdmas = start_step(s)
            if s == 0:
                compute_block((my + 1) % N_DEV)
                compute_block((my - 1) % N_DEV)
            elif s == 1:
                compute_block((my + 2) % N_DEV)
            for rdma in rdmas:
                rdma.wait()
            for d, tgt, hf in dirs:
                recv_c = (my - d * (s + 1)) % N_DEV
                comm_ref[hf, rs] = (
                    comm_ref[hf, rs].astype(jnp.float32)
                    + out_ref[blk(recv_c), cols(hf)]
                ).astype(jnp.bfloat16)

        own_slot = (N_DEV - 1) % 2
        for d, tgt, hf in dirs:
            out_ref[blk((my + d) % N_DEV), cols(hf)] = comm_ref[
                hf, own_slot].astype(jnp.float32)

        for s in range(N_DEV - 1):
            t = (N_DEV - 1) + s
            rs = (t + 1) % 2
            rdmas = start_step(t)
            for rdma in rdmas:
                rdma.wait()
            for d, tgt, hf in dirs:
                recv_c = (my - d * s) % N_DEV
                out_ref[blk(recv_c), cols(hf)] = comm_ref[hf, rs].astype(
                    jnp.float32)

    return pl.pallas_call(
        body,
        out_shape=jax.ShapeDtypeStruct((m, n), jnp.float32),
        in_specs=[
            pl.BlockSpec(memory_space=pltpu.VMEM),
            pl.BlockSpec(memory_space=pl.ANY),
            pl.BlockSpec(memory_space=pl.ANY),
        ],
        out_specs=pl.BlockSpec(memory_space=pltpu.VMEM),
        scratch_shapes=[
            pltpu.VMEM((m, k_in), jnp.bfloat16),
            pltpu.VMEM((m, h_per), jnp.bfloat16),
            pltpu.VMEM((h_per, n), jnp.bfloat16),
            pltpu.VMEM((2, k_in, KC), jnp.float32),
            pltpu.VMEM((2, KC, n), jnp.float32),
            pltpu.SemaphoreType.DMA((2, 2)),
            pltpu.VMEM((2, 2, cm, n2), jnp.bfloat16),
            pltpu.SemaphoreType.DMA((2, 2)),
            pltpu.SemaphoreType.DMA((2, 2)),
        ],
        compiler_params=pltpu.CompilerParams(
            collective_id=0,
            vmem_limit_bytes=60 * 1024 * 1024,
        ),
    )(x, W1, W2)
